# Initial kernel scaffold; baseline (speedup 1.0000x reference)
import jax
import jax.numpy as jnp
from jax.experimental import pallas as pl


def kernel(hidden_states, wg, shared_gate_w, shared_up_w, shared_down_w, w1, w2, w3):
    raise NotImplementedError("write your pallas kernel here")


# speedup vs baseline: 1.3721x; 1.3721x over previous
import functools

import jax
import jax.numpy as jnp
from jax.experimental import pallas as pl
from jax.experimental.pallas import tpu as pltpu

S, D, E, F, FS = 2048, 2048, 8, 1024, 2048
T = 256
MAXB = (S * 2) // T + E - 1
P = MAXB * T
POUT = (MAXB + 1) * T
NTOK = S


def _router_body(x_ref, wg_ref, e1_ref, e2_ref, w1_ref, w2_ref):
    xb = x_ref[...].astype(jnp.bfloat16)
    wgb = wg_ref[...].astype(jnp.bfloat16)
    logits = jax.lax.dot_general(xb, wgb, (((1,), (1,)), ((), ())),
                                 preferred_element_type=jnp.float32)
    m = jnp.max(logits, axis=-1, keepdims=True)
    ex = jnp.exp(logits - m)
    g = ex / jnp.sum(ex, axis=-1, keepdims=True)
    iota8 = jax.lax.broadcasted_iota(jnp.int32, g.shape, 1)
    g1 = jnp.max(g, axis=-1)
    i1 = jnp.min(jnp.where(g == g1[:, None], iota8, E), axis=-1)
    gm = jnp.where(iota8 == i1[:, None], -jnp.inf, g)
    g2 = jnp.max(gm, axis=-1)
    i2 = jnp.min(jnp.where(gm == g2[:, None], iota8, E), axis=-1)
    s = g1 + g2
    e1_ref[...] = i1
    e2_ref[...] = i2
    w1_ref[...] = g1 / s
    w2_ref[...] = g2 / s


def _router(x, wg):
    bt = 256
    return pl.pallas_call(
        _router_body,
        grid=(S // bt,),
        in_specs=[
            pl.BlockSpec((bt, D), lambda t: (t, 0)),
            pl.BlockSpec((E, D), lambda t: (0, 0)),
        ],
        out_specs=[
            pl.BlockSpec((bt,), lambda t: (t,)),
            pl.BlockSpec((bt,), lambda t: (t,)),
            pl.BlockSpec((bt,), lambda t: (t,)),
            pl.BlockSpec((bt,), lambda t: (t,)),
        ],
        out_shape=[
            jax.ShapeDtypeStruct((S,), jnp.int32),
            jax.ShapeDtypeStruct((S,), jnp.int32),
            jax.ShapeDtypeStruct((S,), jnp.float32),
            jax.ShapeDtypeStruct((S,), jnp.float32),
        ],
    )(x, wg)


def _s1_body(x_ref, gw_ref, uw_ref, h_ref):
    xb = x_ref[...].astype(jnp.bfloat16)
    a = jax.lax.dot_general(xb, gw_ref[...].astype(jnp.bfloat16),
                            (((1,), (1,)), ((), ())),
                            preferred_element_type=jnp.float32)
    b = jax.lax.dot_general(xb, uw_ref[...].astype(jnp.bfloat16),
                            (((1,), (1,)), ((), ())),
                            preferred_element_type=jnp.float32)
    h_ref[...] = (a * jax.nn.sigmoid(a) * b).astype(jnp.bfloat16)


def _shared_h(x, gw, uw):
    bt, bf = 256, 1024
    return pl.pallas_call(
        _s1_body,
        grid=(FS // bf, S // bt),
        in_specs=[
            pl.BlockSpec((bt, D), lambda f, t: (t, 0)),
            pl.BlockSpec((bf, D), lambda f, t: (f, 0)),
            pl.BlockSpec((bf, D), lambda f, t: (f, 0)),
        ],
        out_specs=pl.BlockSpec((bt, bf), lambda f, t: (t, f)),
        out_shape=jax.ShapeDtypeStruct((S, FS), jnp.bfloat16),
    )(x, gw, uw)


def _s2_body(h_ref, dw_ref, o_ref):
    y = jax.lax.dot_general(h_ref[...], dw_ref[...].astype(jnp.bfloat16),
                            (((1,), (1,)), ((), ())),
                            preferred_element_type=jnp.float32)
    o_ref[...] = y


def _shared_out(h, dw):
    bt = 256
    return pl.pallas_call(
        _s2_body,
        grid=(S // bt,),
        in_specs=[
            pl.BlockSpec((bt, FS), lambda t: (t, 0)),
            pl.BlockSpec((D, FS), lambda t: (0, 0)),
        ],
        out_specs=pl.BlockSpec((bt, D), lambda t: (t, 0)),
        out_shape=jax.ShapeDtypeStruct((S, D), jnp.float32),
    )(h, dw)


def _g1_body(te_ref, ob_ref, tv_ref, xg_ref, w1_ref, w3_ref, h_ref):
    t = pl.program_id(0)

    @pl.when(tv_ref[t] == 1)
    def _():
        xb = xg_ref[...].astype(jnp.bfloat16)
        a = jax.lax.dot_general(xb, w1_ref[0].astype(jnp.bfloat16),
                                (((1,), (1,)), ((), ())),
                                preferred_element_type=jnp.float32)
        b = jax.lax.dot_general(xb, w3_ref[0].astype(jnp.bfloat16),
                                (((1,), (1,)), ((), ())),
                                preferred_element_type=jnp.float32)
        h_ref[...] = (a * jax.nn.sigmoid(a) * b).astype(jnp.bfloat16)


def _grouped_h(xg, w1, w3, te, ob, tv):
    spec = pltpu.PrefetchScalarGridSpec(
        num_scalar_prefetch=3,
        grid=(MAXB,),
        in_specs=[
            pl.BlockSpec((T, D), lambda t, te, ob, tv: (t, 0)),
            pl.BlockSpec((1, F, D), lambda t, te, ob, tv: (te[t], 0, 0)),
            pl.BlockSpec((1, F, D), lambda t, te, ob, tv: (te[t], 0, 0)),
        ],
        out_specs=pl.BlockSpec((T, F), lambda t, te, ob, tv: (t, 0)),
    )
    return pl.pallas_call(
        _g1_body,
        grid_spec=spec,
        out_shape=jax.ShapeDtypeStruct((P, F), jnp.bfloat16),
    )(te, ob, tv, xg, w1, w3)


def _g2_body(te_ref, ob_ref, tv_ref, h_ref, w2_ref, ws_ref, y_ref):
    t = pl.program_id(0)

    @pl.when(tv_ref[t] == 1)
    def _():
        y = jax.lax.dot_general(h_ref[...], w2_ref[0].astype(jnp.bfloat16),
                                (((1,), (1,)), ((), ())),
                                preferred_element_type=jnp.float32)
        y_ref[...] = y * ws_ref[0, 0][:, None]


def _grouped_out(h, w2, ws3d, te, ob, tv):
    spec = pltpu.PrefetchScalarGridSpec(
        num_scalar_prefetch=3,
        grid=(MAXB,),
        in_specs=[
            pl.BlockSpec((T, F), lambda t, te, ob, tv: (t, 0)),
            pl.BlockSpec((1, D, F), lambda t, te, ob, tv: (te[t], 0, 0)),
            pl.BlockSpec((1, 1, T), lambda t, te, ob, tv: (t, 0, 0)),
        ],
        out_specs=pl.BlockSpec((T, D), lambda t, te, ob, tv: (ob[t], 0)),
    )
    return pl.pallas_call(
        _g2_body,
        grid_spec=spec,
        out_shape=jax.ShapeDtypeStruct((POUT, D), jnp.float32),
    )(te, ob, tv, h, w2, ws3d)


def _dispatch_plan(e1, e2, wA, wB):
    ef = jnp.stack([e1, e2], axis=1).reshape(-1)
    wf = jnp.stack([wA, wB], axis=1).reshape(-1)
    oh = (ef[:, None] == jnp.arange(E)[None, :]).astype(jnp.int32)
    ranks = jnp.cumsum(oh, axis=0)
    rank = jnp.sum(ranks * oh, axis=1) - 1
    counts = ranks[-1]
    nblk = (counts + T - 1) // T
    ends = jnp.cumsum(nblk)
    startblk = ends - nblk
    ppos = startblk[ef] * T + rank
    tok = jnp.arange(2 * S, dtype=jnp.int32) // 2
    disp = jnp.zeros((P,), jnp.int32).at[ppos].set(
        tok, mode="drop", unique_indices=True)
    ws = jnp.zeros((P,), jnp.float32).at[ppos].set(
        wf, mode="drop", unique_indices=True)
    p0 = ppos[0::2]
    p1 = ppos[1::2]
    tvec = jnp.arange(MAXB, dtype=jnp.int32)
    used = ends[-1]
    te = jnp.minimum(jnp.searchsorted(ends, tvec, side="right"),
                     E - 1).astype(jnp.int32)
    tv = (tvec < used).astype(jnp.int32)
    ob = jnp.where(tv == 1, tvec, MAXB).astype(jnp.int32)
    return disp, ws, p0, p1, te, tv, ob


def kernel(hidden_states, wg, shared_gate_w, shared_up_w, shared_down_w, w1, w2, w3):
    B = hidden_states.shape[0]
    x = hidden_states.reshape(S, D)

    e1, e2, wA, wB = _router(x, wg)
    disp, ws, p0, p1, te, tv, ob = _dispatch_plan(e1, e2, wA, wB)

    hs = _shared_h(x, shared_gate_w, shared_up_w)
    shared = _shared_out(hs, shared_down_w)

    xg = jnp.take(x, disp, axis=0)
    h = _grouped_h(xg, w1, w3, te, ob, tv)
    y = _grouped_out(h, w2, ws.reshape(MAXB, 1, T), te, ob, tv)

    routed = jnp.take(y, p0, axis=0) + jnp.take(y, p1, axis=0)
    return (shared + routed).reshape(B, S, D)
